# Initial kernel scaffold; baseline (speedup 1.0000x reference)
import jax
import jax.numpy as jnp
from jax.experimental import pallas as pl


def kernel(sparse_inputs, embed_inputs, w):
    raise NotImplementedError("write your pallas kernel here")


# speedup vs baseline: 1.0303x; 1.0303x over previous
import functools

import jax
import jax.numpy as jnp
from jax import lax
from jax.experimental import pallas as pl
from jax.experimental.pallas import tpu as pltpu
from jax.experimental.pallas import tpu_sc as plsc

_B = 16384
_F = 26
_D = 16
_NW = 32
_BPW = _B // _NW
_JB = _BPW // 128


def _fo_body(idx_hbm, w_hbm, out_hbm, idx_v, vals_v, acc_v, sem):
    wid = lax.axis_index("s") * 2 + lax.axis_index("c")
    pltpu.sync_copy(idx_hbm.at[wid], idx_v)
    pltpu.async_copy(w_hbm.at[idx_v], vals_v, sem).wait()
    for c in range(_BPW // 16):
        v = vals_v[pl.ds(c * 16, 16)]
        for f in range(1, _F):
            v = v + vals_v[pl.ds(f * _BPW + c * 16, 16)]
        acc_v[pl.ds(c * 16, 16)] = v
    pltpu.sync_copy(acc_v, out_hbm.at[wid])


def _first_order(idx, w_flat):
    fo_kernel = functools.partial(
        pl.kernel,
        out_type=jax.ShapeDtypeStruct((_NW, _BPW), jnp.float32),
        mesh=plsc.VectorSubcoreMesh(core_axis_name="c", subcore_axis_name="s"),
        scratch_types=[
            pltpu.VMEM((_F * _BPW,), jnp.int32),
            pltpu.VMEM((_F * _BPW,), jnp.float32),
            pltpu.VMEM((_BPW,), jnp.float32),
            pltpu.SemaphoreType.DMA,
        ],
    )(_fo_body)
    return fo_kernel(idx, w_flat)


def _so_body(x_ref, fo_ref, m_ref, o_ref):
    x = x_ref[...]
    q = jnp.sum(x * x, axis=1, keepdims=True)
    s = jnp.dot(x, m_ref[...], preferred_element_type=jnp.float32)
    ssq = jnp.sum(s * s, axis=1, keepdims=True)
    o_ref[...] = fo_ref[...] + 0.5 * (ssq - q)


def kernel(sparse_inputs, embed_inputs, w):
    idx = sparse_inputs.T.reshape(_F, _NW, _BPW).transpose(1, 0, 2).reshape(_NW, _F * _BPW)
    fo = _first_order(idx, w.reshape(-1)).reshape(_B, 1)

    x = embed_inputs.reshape(_B, _F * _D)
    m = jnp.tile(jnp.eye(_D, dtype=jnp.float32), (_F, 1))
    blk = 1024
    grid = _B // blk
    return pl.pallas_call(
        _so_body,
        grid=(grid,),
        in_specs=[
            pl.BlockSpec((blk, _F * _D), lambda i: (i, 0)),
            pl.BlockSpec((blk, 1), lambda i: (i, 0)),
            pl.BlockSpec((_F * _D, _D), lambda i: (0, 0)),
        ],
        out_specs=pl.BlockSpec((blk, 1), lambda i: (i, 0)),
        out_shape=jax.ShapeDtypeStruct((_B, 1), jnp.float32),
    )(x, fo, m)
